# Initial kernel scaffold; baseline (speedup 1.0000x reference)
import functools

import jax
import jax.numpy as jnp
from jax.experimental import pallas as pl
from jax.experimental.pallas import tpu as pltpu
from jax.experimental.pallas import tpu_sc as plsc

_N_FIELDS = 26
_VOCAB = 100000
_DIM = 16
_B = 16384
_D = _N_FIELDS * _DIM
_NUM_IDX = _B * _N_FIELDS

_GATHER_W = 128
_BS = 2048


def _sc_gather(flat_tables, flat_idx):
    mesh = plsc.VectorSubcoreMesh(core_axis_name="core", subcore_axis_name="subcore")
    idx2d = flat_idx.reshape(1, _NUM_IDX)

    @functools.partial(
        pl.kernel,
        out_type=jax.ShapeDtypeStruct((_NUM_IDX, _DIM), jnp.float32),
        mesh=mesh,
    )
    def gather_kernel(x_hbm, i_hbm, o_hbm):
        def body(i_vmem, o_vmem):
            pltpu.sync_copy(x_hbm.at[i_vmem.at[0]], o_vmem)

        pltpu.emit_pipeline(
            body,
            grid=(_NUM_IDX // _GATHER_W,),
            in_specs=[pl.BlockSpec((1, _GATHER_W), index_map=lambda i: (0, i))],
            out_specs=[pl.BlockSpec((_GATHER_W, _DIM), index_map=lambda i: (i, 0))],
            core_axis_name=("core", "subcore"),
            dimension_semantics=(pltpu.PARALLEL,),
        )(i_hbm, o_hbm)

    return gather_kernel(flat_tables, idx2d)


def _dense_body(x0_ref, cw_ref, cb_ref, w1_ref, b1_ref, w2_ref, b2_ref,
                w3_ref, b3_ref, wf_ref, bf_ref, out_ref):
    x0 = x0_ref[...]
    x = x0
    for l in range(3):
        w = cw_ref[l, :]
        xw = jnp.sum(x * w[None, :], axis=1, keepdims=True)
        x = x0 * xw + cb_ref[l, :][None, :] + x
    h = x0
    for w_ref, b_ref in ((w1_ref, b1_ref), (w2_ref, b2_ref), (w3_ref, b3_ref)):
        h = jnp.maximum(
            jnp.dot(h, w_ref[...], preferred_element_type=jnp.float32)
            + b_ref[0, :][None, :], 0.0)
    wf = wf_ref[...]
    logit = (jnp.dot(x, wf[:_D, :], preferred_element_type=jnp.float32)
             + jnp.dot(h, wf[_D:, :], preferred_element_type=jnp.float32)
             + bf_ref[0, 0])
    out_ref[...] = jax.nn.sigmoid(logit)


def _dense(x0, cross_w, cross_b, W1, b1, W2, b2, W3, b3, Wf, bf):
    grid = (_B // _BS,)
    return pl.pallas_call(
        _dense_body,
        grid=grid,
        in_specs=[
            pl.BlockSpec((_BS, _D), lambda i: (i, 0)),
            pl.BlockSpec((3, _D), lambda i: (0, 0)),
            pl.BlockSpec((3, _D), lambda i: (0, 0)),
            pl.BlockSpec((_D, 256), lambda i: (0, 0)),
            pl.BlockSpec((1, 256), lambda i: (0, 0)),
            pl.BlockSpec((256, 128), lambda i: (0, 0)),
            pl.BlockSpec((1, 128), lambda i: (0, 0)),
            pl.BlockSpec((128, 64), lambda i: (0, 0)),
            pl.BlockSpec((1, 64), lambda i: (0, 0)),
            pl.BlockSpec((_D + 64, 1), lambda i: (0, 0)),
            pl.BlockSpec((1, 1), lambda i: (0, 0)),
        ],
        out_specs=pl.BlockSpec((_BS, 1), lambda i: (i, 0)),
        out_shape=jax.ShapeDtypeStruct((_B, 1), jnp.float32),
    )(x0, cross_w, cross_b, W1, b1.reshape(1, 256), W2, b2.reshape(1, 128),
      W3, b3.reshape(1, 64), Wf, bf.reshape(1, 1))


def kernel(inputs, tables, cross_w, cross_b, W1, b1, W2, b2, W3, b3, Wf, bf):
    flat_tables = tables.reshape(_N_FIELDS * _VOCAB, _DIM)
    offs = (jnp.arange(_N_FIELDS, dtype=jnp.int32) * _VOCAB)[None, :]
    flat_idx = (inputs + offs).reshape(-1)
    gathered = _sc_gather(flat_tables, flat_idx)
    x0 = gathered.reshape(_B, _D)
    return _dense(x0, cross_w, cross_b, W1, b1, W2, b2, W3, b3, Wf, bf)


# speedup vs baseline: 1.1520x; 1.1520x over previous
import functools

import jax
import jax.numpy as jnp
from jax.experimental import pallas as pl
from jax.experimental.pallas import tpu as pltpu
from jax.experimental.pallas import tpu_sc as plsc

_N_FIELDS = 26
_VOCAB = 100000
_DIM = 16
_B = 16384
_D = _N_FIELDS * _DIM
_NUM_IDX = _B * _N_FIELDS

_GATHER_W = 128
_BS = 2048


def _sc_gather(flat_tables, flat_idx):
    mesh = plsc.VectorSubcoreMesh(core_axis_name="core", subcore_axis_name="subcore")
    idx2d = flat_idx.reshape(1, _NUM_IDX)

    @functools.partial(
        pl.kernel,
        out_type=jax.ShapeDtypeStruct((_NUM_IDX, _DIM), jnp.float32),
        mesh=mesh,
        compiler_params=pltpu.CompilerParams(use_tc_tiling_on_sc=False),
    )
    def gather_kernel(x_hbm, i_hbm, o_hbm):
        def body(i_vmem, o_vmem):
            pltpu.sync_copy(x_hbm.at[i_vmem.at[0]], o_vmem)

        pltpu.emit_pipeline(
            body,
            grid=(_NUM_IDX // _GATHER_W,),
            in_specs=[pl.BlockSpec((1, _GATHER_W), index_map=lambda i: (0, i))],
            out_specs=[pl.BlockSpec((_GATHER_W, _DIM), index_map=lambda i: (i, 0))],
            core_axis_name=("core", "subcore"),
            dimension_semantics=(pltpu.PARALLEL,),
        )(i_hbm, o_hbm)

    return gather_kernel(flat_tables, idx2d)


def _dense_body(x0_ref, cw_ref, cb_ref, w1_ref, b1_ref, w2_ref, b2_ref,
                w3_ref, b3_ref, wf_ref, bf_ref, out_ref):
    x0 = x0_ref[...]
    x = x0
    for l in range(3):
        w = cw_ref[l, :]
        xw = jnp.sum(x * w[None, :], axis=1, keepdims=True)
        x = x0 * xw + cb_ref[l, :][None, :] + x
    h = x0
    for w_ref, b_ref in ((w1_ref, b1_ref), (w2_ref, b2_ref), (w3_ref, b3_ref)):
        h = jnp.maximum(
            jnp.dot(h, w_ref[...], preferred_element_type=jnp.float32)
            + b_ref[0, :][None, :], 0.0)
    wf = wf_ref[...]
    logit = (jnp.dot(x, wf[:_D, :], preferred_element_type=jnp.float32)
             + jnp.dot(h, wf[_D:, :], preferred_element_type=jnp.float32)
             + bf_ref[0, 0])
    out_ref[...] = jax.nn.sigmoid(logit)


def _dense(x0, cross_w, cross_b, W1, b1, W2, b2, W3, b3, Wf, bf):
    grid = (_B // _BS,)
    return pl.pallas_call(
        _dense_body,
        grid=grid,
        in_specs=[
            pl.BlockSpec((_BS, _D), lambda i: (i, 0)),
            pl.BlockSpec((3, _D), lambda i: (0, 0)),
            pl.BlockSpec((3, _D), lambda i: (0, 0)),
            pl.BlockSpec((_D, 256), lambda i: (0, 0)),
            pl.BlockSpec((1, 256), lambda i: (0, 0)),
            pl.BlockSpec((256, 128), lambda i: (0, 0)),
            pl.BlockSpec((1, 128), lambda i: (0, 0)),
            pl.BlockSpec((128, 64), lambda i: (0, 0)),
            pl.BlockSpec((1, 64), lambda i: (0, 0)),
            pl.BlockSpec((_D + 64, 1), lambda i: (0, 0)),
            pl.BlockSpec((1, 1), lambda i: (0, 0)),
        ],
        out_specs=pl.BlockSpec((_BS, 1), lambda i: (i, 0)),
        out_shape=jax.ShapeDtypeStruct((_B, 1), jnp.float32),
    )(x0, cross_w, cross_b, W1, b1.reshape(1, 256), W2, b2.reshape(1, 128),
      W3, b3.reshape(1, 64), Wf, bf.reshape(1, 1))


def kernel(inputs, tables, cross_w, cross_b, W1, b1, W2, b2, W3, b3, Wf, bf):
    flat_tables = tables.reshape(_N_FIELDS * _VOCAB, _DIM)
    offs = (jnp.arange(_N_FIELDS, dtype=jnp.int32) * _VOCAB)[None, :]
    flat_idx = (inputs + offs).reshape(-1)
    gathered = _sc_gather(flat_tables, flat_idx)
    x0 = gathered.reshape(_B, _D)
    return _dense(x0, cross_w, cross_b, W1, b1, W2, b2, W3, b3, Wf, bf)


# speedup vs baseline: 1.4566x; 1.2644x over previous
import functools

import jax
import jax.numpy as jnp
from jax.experimental import pallas as pl
from jax.experimental.pallas import tpu as pltpu
from jax.experimental.pallas import tpu_sc as plsc

_N_FIELDS = 26
_VOCAB = 100000
_DIM = 16
_B = 16384
_D = _N_FIELDS * _DIM
_NUM_IDX = _B * _N_FIELDS

_GATHER_W = 128
_BS = 2048


def _format_body(t_ref, o_ref):
    chunk = 4992
    rows = chunk * _DIM // 128
    n_c = _VOCAB // chunk
    rem = _VOCAB - n_c * chunk
    rem_rows = rem * _DIM // 128
    f_rows = _VOCAB * _DIM // 128

    def pack(x, nv):
        y3 = x.T.reshape(nv // 8, 8, _DIM)
        return jnp.concatenate([y3[:, k, :] for k in range(8)], axis=1)

    for h in range(2):
        def step(c, carry, h=h):
            x = t_ref[h, :, pl.ds(c * chunk, chunk)]
            o_ref[0, pl.ds(h * f_rows + c * rows, rows), :] = pack(x, chunk)
            return carry

        jax.lax.fori_loop(0, n_c, step, 0)
        xr = t_ref[h, :, pl.ds(n_c * chunk, rem)]
        o_ref[0, pl.ds(h * f_rows + n_c * rows, rem_rows), :] = pack(xr, rem)


def _format_tables(tables_t):
    return pl.pallas_call(
        _format_body,
        grid=(_N_FIELDS // 2,),
        in_specs=[pl.BlockSpec((2, _DIM, _VOCAB), lambda f: (f, 0, 0))],
        out_specs=pl.BlockSpec((1, 25000, 128), lambda f: (f, 0, 0)),
        out_shape=jax.ShapeDtypeStruct((13, 25000, 128), jnp.float32),
    )(tables_t)


def _sc_gather(flat_tables, flat_idx):
    mesh = plsc.VectorSubcoreMesh(core_axis_name="core", subcore_axis_name="subcore")
    idx2d = flat_idx.reshape(1, _NUM_IDX)

    @functools.partial(
        pl.kernel,
        out_type=jax.ShapeDtypeStruct((_NUM_IDX, _DIM), jnp.float32),
        mesh=mesh,
        compiler_params=pltpu.CompilerParams(use_tc_tiling_on_sc=False),
    )
    def gather_kernel(x_hbm, i_hbm, o_hbm):
        def body(i_vmem, o_vmem):
            pltpu.sync_copy(x_hbm.at[i_vmem.at[0]], o_vmem)

        pltpu.emit_pipeline(
            body,
            grid=(_NUM_IDX // _GATHER_W,),
            in_specs=[pl.BlockSpec((1, _GATHER_W), index_map=lambda i: (0, i))],
            out_specs=[pl.BlockSpec((_GATHER_W, _DIM), index_map=lambda i: (i, 0))],
            core_axis_name=("core", "subcore"),
            dimension_semantics=(pltpu.PARALLEL,),
        )(i_hbm, o_hbm)

    return gather_kernel(flat_tables, idx2d)


def _dense_body(x0_ref, cw_ref, cb_ref, w1_ref, b1_ref, w2_ref, b2_ref,
                w3_ref, b3_ref, wf_ref, bf_ref, out_ref):
    x0 = x0_ref[...]
    x = x0
    for l in range(3):
        w = cw_ref[l, :]
        xw = jnp.sum(x * w[None, :], axis=1, keepdims=True)
        x = x0 * xw + cb_ref[l, :][None, :] + x
    h = x0
    for w_ref, b_ref in ((w1_ref, b1_ref), (w2_ref, b2_ref), (w3_ref, b3_ref)):
        h = jnp.maximum(
            jnp.dot(h, w_ref[...], preferred_element_type=jnp.float32)
            + b_ref[0, :][None, :], 0.0)
    wf = wf_ref[...]
    logit = (jnp.dot(x, wf[:_D, :], preferred_element_type=jnp.float32)
             + jnp.dot(h, wf[_D:, :], preferred_element_type=jnp.float32)
             + bf_ref[0, 0])
    out_ref[...] = jax.nn.sigmoid(logit)


def _dense(x0, cross_w, cross_b, W1, b1, W2, b2, W3, b3, Wf, bf):
    grid = (_B // _BS,)
    return pl.pallas_call(
        _dense_body,
        grid=grid,
        in_specs=[
            pl.BlockSpec((_BS, _D), lambda i: (i, 0)),
            pl.BlockSpec((3, _D), lambda i: (0, 0)),
            pl.BlockSpec((3, _D), lambda i: (0, 0)),
            pl.BlockSpec((_D, 256), lambda i: (0, 0)),
            pl.BlockSpec((1, 256), lambda i: (0, 0)),
            pl.BlockSpec((256, 128), lambda i: (0, 0)),
            pl.BlockSpec((1, 128), lambda i: (0, 0)),
            pl.BlockSpec((128, 64), lambda i: (0, 0)),
            pl.BlockSpec((1, 64), lambda i: (0, 0)),
            pl.BlockSpec((_D + 64, 1), lambda i: (0, 0)),
            pl.BlockSpec((1, 1), lambda i: (0, 0)),
        ],
        out_specs=pl.BlockSpec((_BS, 1), lambda i: (i, 0)),
        out_shape=jax.ShapeDtypeStruct((_B, 1), jnp.float32),
    )(x0, cross_w, cross_b, W1, b1.reshape(1, 256), W2, b2.reshape(1, 128),
      W3, b3.reshape(1, 64), Wf, bf.reshape(1, 1))


def kernel(inputs, tables, cross_w, cross_b, W1, b1, W2, b2, W3, b3, Wf, bf):
    tables_t = jnp.transpose(tables, (0, 2, 1))
    flat_tables = _format_tables(tables_t).reshape(_N_FIELDS * _VOCAB, _DIM)
    offs = (jnp.arange(_N_FIELDS, dtype=jnp.int32) * _VOCAB)[None, :]
    flat_idx = (inputs + offs).reshape(-1)
    gathered = _sc_gather(flat_tables, flat_idx)
    x0 = gathered.reshape(_B, _D)
    return _dense(x0, cross_w, cross_b, W1, b1, W2, b2, W3, b3, Wf, bf)


# speedup vs baseline: 4.4691x; 3.0682x over previous
import functools

import jax
import jax.numpy as jnp
from jax.experimental import pallas as pl
from jax.experimental.pallas import tpu as pltpu
from jax.experimental.pallas import tpu_sc as plsc

_N_FIELDS = 26
_VOCAB = 100000
_DIM = 16
_B = 16384
_D = _N_FIELDS * _DIM
_NUM_IDX = _B * _N_FIELDS

_GATHER_W = 128
_BS = 2048


def _format_body(t_ref, o_ref):
    chunk = 4096
    sub = chunk // 8
    rows = chunk * _DIM // 128
    n_c = 98304 // chunk
    rem = _VOCAB - n_c * chunk
    rem_rows = rem * _DIM // 128
    f_rows = _VOCAB * _DIM // 128

    for h in range(2):
        def step(c, carry, h=h):
            x = t_ref[h, :, pl.ds(c * chunk, chunk)]
            xx = jnp.concatenate(
                [x[:, j * sub:(j + 1) * sub] for j in range(8)], axis=0)
            o_ref[0, pl.ds(h * f_rows + c * rows, rows), :] = xx.T
            return carry

        jax.lax.fori_loop(0, n_c, step, 0)
        xr = t_ref[h, :, pl.ds(n_c * chunk, rem)]
        y3 = xr.T.reshape(rem // 8, 8, _DIM)
        packed = jnp.concatenate([y3[:, k, :] for k in range(8)], axis=1)
        o_ref[0, pl.ds(h * f_rows + n_c * rows, rem_rows), :] = packed


def _format_tables(tables_t):
    return pl.pallas_call(
        _format_body,
        grid=(_N_FIELDS // 2,),
        in_specs=[pl.BlockSpec((2, _DIM, _VOCAB), lambda f: (f, 0, 0))],
        out_specs=pl.BlockSpec((1, 25000, 128), lambda f: (f, 0, 0)),
        out_shape=jax.ShapeDtypeStruct((13, 25000, 128), jnp.float32),
    )(tables_t)


def _sc_gather(flat_tables, flat_idx):
    mesh = plsc.VectorSubcoreMesh(core_axis_name="core", subcore_axis_name="subcore")
    idx2d = flat_idx.reshape(1, _NUM_IDX)

    @functools.partial(
        pl.kernel,
        out_type=jax.ShapeDtypeStruct((_NUM_IDX, _DIM), jnp.float32),
        mesh=mesh,
        compiler_params=pltpu.CompilerParams(use_tc_tiling_on_sc=False),
    )
    def gather_kernel(x_hbm, i_hbm, o_hbm):
        def body(i_vmem, o_vmem):
            pltpu.sync_copy(x_hbm.at[i_vmem.at[0]], o_vmem)

        pltpu.emit_pipeline(
            body,
            grid=(_NUM_IDX // _GATHER_W,),
            in_specs=[pl.BlockSpec((1, _GATHER_W), index_map=lambda i: (0, i))],
            out_specs=[pl.BlockSpec((_GATHER_W, _DIM), index_map=lambda i: (i, 0))],
            core_axis_name=("core", "subcore"),
            dimension_semantics=(pltpu.PARALLEL,),
        )(i_hbm, o_hbm)

    return gather_kernel(flat_tables, idx2d)


def _dense_body(x0_ref, cw_ref, cb_ref, w1_ref, b1_ref, w2_ref, b2_ref,
                w3_ref, b3_ref, wf_ref, bf_ref, out_ref):
    x0 = x0_ref[...]
    x = x0
    for l in range(3):
        w = cw_ref[l, :]
        xw = jnp.sum(x * w[None, :], axis=1, keepdims=True)
        x = x0 * xw + cb_ref[l, :][None, :] + x
    h = x0
    for w_ref, b_ref in ((w1_ref, b1_ref), (w2_ref, b2_ref), (w3_ref, b3_ref)):
        h = jnp.maximum(
            jnp.dot(h, w_ref[...], preferred_element_type=jnp.float32)
            + b_ref[0, :][None, :], 0.0)
    wf = wf_ref[...]
    logit = (jnp.dot(x, wf[:_D, :], preferred_element_type=jnp.float32)
             + jnp.dot(h, wf[_D:, :], preferred_element_type=jnp.float32)
             + bf_ref[0, 0])
    out_ref[...] = jax.nn.sigmoid(logit)


def _dense(x0, cross_w, cross_b, W1, b1, W2, b2, W3, b3, Wf, bf):
    grid = (_B // _BS,)
    return pl.pallas_call(
        _dense_body,
        grid=grid,
        in_specs=[
            pl.BlockSpec((_BS, _D), lambda i: (i, 0)),
            pl.BlockSpec((3, _D), lambda i: (0, 0)),
            pl.BlockSpec((3, _D), lambda i: (0, 0)),
            pl.BlockSpec((_D, 256), lambda i: (0, 0)),
            pl.BlockSpec((1, 256), lambda i: (0, 0)),
            pl.BlockSpec((256, 128), lambda i: (0, 0)),
            pl.BlockSpec((1, 128), lambda i: (0, 0)),
            pl.BlockSpec((128, 64), lambda i: (0, 0)),
            pl.BlockSpec((1, 64), lambda i: (0, 0)),
            pl.BlockSpec((_D + 64, 1), lambda i: (0, 0)),
            pl.BlockSpec((1, 1), lambda i: (0, 0)),
        ],
        out_specs=pl.BlockSpec((_BS, 1), lambda i: (i, 0)),
        out_shape=jax.ShapeDtypeStruct((_B, 1), jnp.float32),
    )(x0, cross_w, cross_b, W1, b1.reshape(1, 256), W2, b2.reshape(1, 128),
      W3, b3.reshape(1, 64), Wf, bf.reshape(1, 1))


def kernel(inputs, tables, cross_w, cross_b, W1, b1, W2, b2, W3, b3, Wf, bf):
    tables_t = jnp.transpose(tables, (0, 2, 1))
    flat_tables = _format_tables(tables_t).reshape(_N_FIELDS * _VOCAB, _DIM)
    offs = (jnp.arange(_N_FIELDS, dtype=jnp.int32) * _VOCAB)[None, :]
    v = inputs
    g = jnp.where(
        v < 98304,
        (v & ~4095) | ((v & 511) << 3) | ((v >> 9) & 7),
        v)
    flat_idx = (g + offs).reshape(-1)
    gathered = _sc_gather(flat_tables, flat_idx)
    x0 = gathered.reshape(_B, _D)
    return _dense(x0, cross_w, cross_b, W1, b1, W2, b2, W3, b3, Wf, bf)


# speedup vs baseline: 4.4753x; 1.0014x over previous
import functools

import jax
import jax.numpy as jnp
from jax.experimental import pallas as pl
from jax.experimental.pallas import tpu as pltpu
from jax.experimental.pallas import tpu_sc as plsc

_N_FIELDS = 26
_VOCAB = 100000
_DIM = 16
_B = 16384
_D = _N_FIELDS * _DIM
_NUM_IDX = _B * _N_FIELDS

_GATHER_W = 128
_BS = 2048


def _format_body(t_ref, o_ref):
    chunk = 4096
    sub = chunk // 8
    rows = chunk * _DIM // 128
    n_c = 98304 // chunk
    rem = _VOCAB - n_c * chunk
    rem_rows = rem * _DIM // 128
    f_rows = _VOCAB * _DIM // 128

    for h in range(2):
        def step(c, carry, h=h):
            x = t_ref[h, :, pl.ds(c * chunk, chunk)]
            xx = jnp.concatenate(
                [x[:, j * sub:(j + 1) * sub] for j in range(8)], axis=0)
            o_ref[0, pl.ds(h * f_rows + c * rows, rows), :] = xx.T
            return carry

        jax.lax.fori_loop(0, n_c, step, 0)
        xr = t_ref[h, :, pl.ds(n_c * chunk, rem)]
        y3 = xr.T.reshape(rem // 8, 8, _DIM)
        packed = jnp.concatenate([y3[:, k, :] for k in range(8)], axis=1)
        o_ref[0, pl.ds(h * f_rows + n_c * rows, rem_rows), :] = packed


def _format_tables(tables_t):
    return pl.pallas_call(
        _format_body,
        grid=(_N_FIELDS // 2,),
        in_specs=[pl.BlockSpec((2, _DIM, _VOCAB), lambda f: (f, 0, 0))],
        out_specs=pl.BlockSpec((1, 25000, 128), lambda f: (f, 0, 0)),
        out_shape=jax.ShapeDtypeStruct((13, 25000, 128), jnp.float32),
        compiler_params=pltpu.CompilerParams(
            dimension_semantics=("parallel",)),
    )(tables_t)


def _sc_gather(flat_tables, flat_idx):
    mesh = plsc.VectorSubcoreMesh(core_axis_name="core", subcore_axis_name="subcore")
    idx2d = flat_idx.reshape(1, _NUM_IDX)

    @functools.partial(
        pl.kernel,
        out_type=jax.ShapeDtypeStruct((_NUM_IDX, _DIM), jnp.float32),
        mesh=mesh,
        compiler_params=pltpu.CompilerParams(use_tc_tiling_on_sc=False),
    )
    def gather_kernel(x_hbm, i_hbm, o_hbm):
        def body(i_vmem, o_vmem):
            pltpu.sync_copy(x_hbm.at[i_vmem.at[0]], o_vmem)

        pltpu.emit_pipeline(
            body,
            grid=(_NUM_IDX // _GATHER_W,),
            in_specs=[pl.BlockSpec((1, _GATHER_W), index_map=lambda i: (0, i))],
            out_specs=[pl.BlockSpec((_GATHER_W, _DIM), index_map=lambda i: (i, 0))],
            core_axis_name=("core", "subcore"),
            dimension_semantics=(pltpu.PARALLEL,),
        )(i_hbm, o_hbm)

    return gather_kernel(flat_tables, idx2d)


def _dense_body(x0_ref, cw_ref, cb_ref, w1_ref, b1_ref, w2_ref, b2_ref,
                w3_ref, b3_ref, wf_ref, bf_ref, out_ref):
    x0 = x0_ref[...]
    x = x0
    for l in range(3):
        w = cw_ref[l, :]
        xw = jnp.sum(x * w[None, :], axis=1, keepdims=True)
        x = x0 * xw + cb_ref[l, :][None, :] + x
    h = x0
    for w_ref, b_ref in ((w1_ref, b1_ref), (w2_ref, b2_ref), (w3_ref, b3_ref)):
        h = jnp.maximum(
            jnp.dot(h, w_ref[...], preferred_element_type=jnp.float32)
            + b_ref[0, :][None, :], 0.0)
    wf = wf_ref[...]
    logit = (jnp.dot(x, wf[:_D, :], preferred_element_type=jnp.float32)
             + jnp.dot(h, wf[_D:, :], preferred_element_type=jnp.float32)
             + bf_ref[0, 0])
    out_ref[...] = jax.nn.sigmoid(logit)


def _dense(x0, cross_w, cross_b, W1, b1, W2, b2, W3, b3, Wf, bf):
    grid = (_B // _BS,)
    return pl.pallas_call(
        _dense_body,
        grid=grid,
        in_specs=[
            pl.BlockSpec((_BS, _D), lambda i: (i, 0)),
            pl.BlockSpec((3, _D), lambda i: (0, 0)),
            pl.BlockSpec((3, _D), lambda i: (0, 0)),
            pl.BlockSpec((_D, 256), lambda i: (0, 0)),
            pl.BlockSpec((1, 256), lambda i: (0, 0)),
            pl.BlockSpec((256, 128), lambda i: (0, 0)),
            pl.BlockSpec((1, 128), lambda i: (0, 0)),
            pl.BlockSpec((128, 64), lambda i: (0, 0)),
            pl.BlockSpec((1, 64), lambda i: (0, 0)),
            pl.BlockSpec((_D + 64, 1), lambda i: (0, 0)),
            pl.BlockSpec((1, 1), lambda i: (0, 0)),
        ],
        out_specs=pl.BlockSpec((_BS, 1), lambda i: (i, 0)),
        out_shape=jax.ShapeDtypeStruct((_B, 1), jnp.float32),
        compiler_params=pltpu.CompilerParams(
            dimension_semantics=("parallel",)),
    )(x0, cross_w, cross_b, W1, b1.reshape(1, 256), W2, b2.reshape(1, 128),
      W3, b3.reshape(1, 64), Wf, bf.reshape(1, 1))


def kernel(inputs, tables, cross_w, cross_b, W1, b1, W2, b2, W3, b3, Wf, bf):
    tables_t = jnp.transpose(tables, (0, 2, 1))
    flat_tables = _format_tables(tables_t).reshape(_N_FIELDS * _VOCAB, _DIM)
    offs = (jnp.arange(_N_FIELDS, dtype=jnp.int32) * _VOCAB)[None, :]
    v = inputs
    g = jnp.where(
        v < 98304,
        (v & ~4095) | ((v & 511) << 3) | ((v >> 9) & 7),
        v)
    flat_idx = (g + offs).reshape(-1)
    gathered = _sc_gather(flat_tables, flat_idx)
    x0 = gathered.reshape(_B, _D)
    return _dense(x0, cross_w, cross_b, W1, b1, W2, b2, W3, b3, Wf, bf)


# speedup vs baseline: 4.8413x; 1.0818x over previous
import functools

import jax
import jax.numpy as jnp
from jax.experimental import pallas as pl
from jax.experimental.pallas import tpu as pltpu
from jax.experimental.pallas import tpu_sc as plsc

_N_FIELDS = 26
_VOCAB = 100000
_DIM = 16
_B = 16384
_D = _N_FIELDS * _DIM
_NUM_IDX = _B * _N_FIELDS

_GATHER_W = 128
_BS = 2048


def _format_body(t_ref, o_ref):
    chunk = 4096
    sub = chunk // 8
    rows = chunk * _DIM // 128
    n_c = 98304 // chunk
    rem = _VOCAB - n_c * chunk
    rem_rows = rem * _DIM // 128
    f_rows = _VOCAB * _DIM // 128

    for h in range(2):
        def step(c, carry, h=h):
            x = t_ref[h, :, pl.ds(c * chunk, chunk)]
            xx = jnp.concatenate(
                [x[:, j * sub:(j + 1) * sub] for j in range(8)], axis=0)
            o_ref[0, pl.ds(h * f_rows + c * rows, rows), :] = xx.T
            return carry

        jax.lax.fori_loop(0, n_c, step, 0)
        xr = t_ref[h, :, pl.ds(n_c * chunk, rem)]
        y3 = xr.T.reshape(rem // 8, 8, _DIM)
        packed = jnp.concatenate([y3[:, k, :] for k in range(8)], axis=1)
        o_ref[0, pl.ds(h * f_rows + n_c * rows, rem_rows), :] = packed


def _format_tables(tables_t, pair_off, n_pairs):
    return pl.pallas_call(
        _format_body,
        grid=(n_pairs,),
        in_specs=[pl.BlockSpec((2, _DIM, _VOCAB),
                               lambda f: (f + pair_off, 0, 0))],
        out_specs=pl.BlockSpec((1, 25000, 128), lambda f: (f, 0, 0)),
        out_shape=jax.ShapeDtypeStruct((n_pairs, 25000, 128), jnp.float32),
        compiler_params=pltpu.CompilerParams(
            dimension_semantics=("parallel",)),
    )(tables_t)


def _sc_gather(flat_tables, flat_idx, num_idx):
    mesh = plsc.VectorSubcoreMesh(core_axis_name="core", subcore_axis_name="subcore")
    idx2d = flat_idx.reshape(1, num_idx)

    @functools.partial(
        pl.kernel,
        out_type=jax.ShapeDtypeStruct((num_idx, _DIM), jnp.float32),
        mesh=mesh,
        compiler_params=pltpu.CompilerParams(use_tc_tiling_on_sc=False),
    )
    def gather_kernel(x_hbm, i_hbm, o_hbm):
        def body(i_vmem, o_vmem):
            pltpu.sync_copy(x_hbm.at[i_vmem.at[0]], o_vmem)

        pltpu.emit_pipeline(
            body,
            grid=(num_idx // _GATHER_W,),
            in_specs=[pl.BlockSpec((1, _GATHER_W), index_map=lambda i: (0, i))],
            out_specs=[pl.BlockSpec((_GATHER_W, _DIM), index_map=lambda i: (i, 0))],
            core_axis_name=("core", "subcore"),
            dimension_semantics=(pltpu.PARALLEL,),
        )(i_hbm, o_hbm)

    return gather_kernel(flat_tables, idx2d)


def _dense_body(x0a_ref, x0b_ref, cw_ref, cb_ref, w1_ref, b1_ref, w2_ref, b2_ref,
                w3_ref, b3_ref, wf_ref, bf_ref, out_ref):
    x0 = jnp.concatenate([x0a_ref[...], x0b_ref[...]], axis=1)
    x = x0
    for l in range(3):
        w = cw_ref[l, :]
        xw = jnp.sum(x * w[None, :], axis=1, keepdims=True)
        x = x0 * xw + cb_ref[l, :][None, :] + x
    h = x0
    for w_ref, b_ref in ((w1_ref, b1_ref), (w2_ref, b2_ref), (w3_ref, b3_ref)):
        h = jnp.maximum(
            jnp.dot(h, w_ref[...], preferred_element_type=jnp.float32)
            + b_ref[0, :][None, :], 0.0)
    wf = wf_ref[...]
    logit = (jnp.dot(x, wf[:_D, :], preferred_element_type=jnp.float32)
             + jnp.dot(h, wf[_D:, :], preferred_element_type=jnp.float32)
             + bf_ref[0, 0])
    out_ref[...] = jax.nn.sigmoid(logit)


def _dense(x0a, x0b, cross_w, cross_b, W1, b1, W2, b2, W3, b3, Wf, bf):
    grid = (_B // _BS,)
    return pl.pallas_call(
        _dense_body,
        grid=grid,
        in_specs=[
            pl.BlockSpec((_BS, x0a.shape[1]), lambda i: (i, 0)),
            pl.BlockSpec((_BS, x0b.shape[1]), lambda i: (i, 0)),
            pl.BlockSpec((3, _D), lambda i: (0, 0)),
            pl.BlockSpec((3, _D), lambda i: (0, 0)),
            pl.BlockSpec((_D, 256), lambda i: (0, 0)),
            pl.BlockSpec((1, 256), lambda i: (0, 0)),
            pl.BlockSpec((256, 128), lambda i: (0, 0)),
            pl.BlockSpec((1, 128), lambda i: (0, 0)),
            pl.BlockSpec((128, 64), lambda i: (0, 0)),
            pl.BlockSpec((1, 64), lambda i: (0, 0)),
            pl.BlockSpec((_D + 64, 1), lambda i: (0, 0)),
            pl.BlockSpec((1, 1), lambda i: (0, 0)),
        ],
        out_specs=pl.BlockSpec((_BS, 1), lambda i: (i, 0)),
        out_shape=jax.ShapeDtypeStruct((_B, 1), jnp.float32),
        compiler_params=pltpu.CompilerParams(
            dimension_semantics=("parallel",)),
    )(x0a, x0b, cross_w, cross_b, W1, b1.reshape(1, 256), W2, b2.reshape(1, 128),
      W3, b3.reshape(1, 64), Wf, bf.reshape(1, 1))


def kernel(inputs, tables, cross_w, cross_b, W1, b1, W2, b2, W3, b3, Wf, bf):
    tables_t = jnp.transpose(tables, (0, 2, 1))
    n_fa = 14
    n_fb = _N_FIELDS - n_fa
    flat_a = _format_tables(tables_t, 0, n_fa // 2).reshape(n_fa * _VOCAB, _DIM)
    flat_b = _format_tables(tables_t, n_fa // 2, n_fb // 2).reshape(n_fb * _VOCAB, _DIM)
    v = inputs
    g = jnp.where(
        v < 98304,
        (v & ~4095) | ((v & 511) << 3) | ((v >> 9) & 7),
        v)
    offs = (jnp.arange(_N_FIELDS, dtype=jnp.int32) * _VOCAB)[None, :]
    idx_a = (g[:, :n_fa] + offs[:, :n_fa]).reshape(-1)
    idx_b = (g[:, n_fa:] + offs[:, :n_fb]).reshape(-1)
    ga = _sc_gather(flat_a, idx_a, _B * n_fa)
    gb = _sc_gather(flat_b, idx_b, _B * n_fb)
    x0a = ga.reshape(_B, n_fa * _DIM)
    x0b = gb.reshape(_B, n_fb * _DIM)
    return _dense(x0a, x0b, cross_w, cross_b, W1, b1, W2, b2, W3, b3, Wf, bf)
